# Initial kernel scaffold; baseline (speedup 1.0000x reference)
import jax
import jax.numpy as jnp
from jax.experimental import pallas as pl


def kernel(x, edge_index, W0, b0, W1, b1, W2, b2, Wl, bl):
    raise NotImplementedError("write your pallas kernel here")


# speedup vs baseline: 10.1643x; 10.1643x over previous
import functools

import jax
import jax.numpy as jnp
from jax import lax
from jax.experimental import pallas as pl
from jax.experimental.pallas import tpu as pltpu
from jax.experimental.pallas import tpu_sc as plsc

N = 10000
F = 128
E = 320000
NC, NS = 2, 16
B = 80
EPT = E // (NC * NS)
IT = EPT // B
RPT = 624
TAIL = N - NS * RPT

_MESH = dict(core_axis_name="c", subcore_axis_name="s", num_cores=NC,
             num_subcores=NS)


def _edge_body(hs, srcr, dstr, out, src_v, dst_v, rows_v, z_v, acc, sem):
    c = lax.axis_index("c")
    s = lax.axis_index("s")

    def zb(j, carry):
        for k in range(8):
            z_v[j, pl.ds(k * 16, 16)] = jnp.zeros((16,), jnp.float32)
        return carry
    lax.fori_loop(0, 16, zb, 0)

    def zc(r, carry):
        pltpu.sync_copy(z_v, acc.at[pl.ds(s * RPT + r * 16, 16)])
        return carry
    lax.fori_loop(0, RPT // 16, zc, 0)

    @pl.when(s == NS - 1)
    def _():
        pltpu.sync_copy(z_v, acc.at[pl.ds(NS * RPT, TAIL)])

    plsc.subcore_barrier()

    def step(i, carry):
        pltpu.sync_copy(srcr.at[c, s, i], src_v)
        pltpu.async_copy(hs.at[src_v.at[0]], rows_v, sem).wait()
        pltpu.sync_copy(dstr.at[c, s, i], dst_v)
        pltpu.sync_copy(rows_v, acc.at[dst_v.at[0]], add=True)
        return carry
    lax.fori_loop(0, IT, step, 0)

    plsc.subcore_barrier()

    def co(r, carry):
        pltpu.sync_copy(acc.at[pl.ds(s * RPT + r * 16, 16)], z_v)
        pltpu.sync_copy(z_v, out.at[c, pl.ds(s * RPT + r * 16, 16)])
        return carry
    lax.fori_loop(0, RPT // 16, co, 0)

    @pl.when(s == NS - 1)
    def _():
        pltpu.sync_copy(acc.at[pl.ds(NS * RPT, TAIL)], z_v)
        pltpu.sync_copy(z_v, out.at[c, pl.ds(NS * RPT, TAIL)])


_edge_pass = pl.kernel(
    _edge_body,
    out_type=jax.ShapeDtypeStruct((NC, N, F), jnp.float32),
    mesh=plsc.VectorSubcoreMesh(**_MESH),
    scratch_types=[
        pltpu.VMEM((1, B), jnp.int32),
        pltpu.VMEM((1, B), jnp.int32),
        pltpu.VMEM((B, F), jnp.float32),
        pltpu.VMEM((16, F), jnp.float32),
        pltpu.VMEM_SHARED((N, F), jnp.float32),
        pltpu.SemaphoreType.DMA,
    ],
)


def _deg_body(dstr, out, dst_v, ones_v, z_v, acc):
    c = lax.axis_index("c")
    s = lax.axis_index("s")

    def zb(j, carry):
        for k in range(8):
            z_v[j, pl.ds(k * 16, 16)] = jnp.zeros((16,), jnp.float32)
        return carry
    lax.fori_loop(0, 16, zb, 0)

    def zc(r, carry):
        pltpu.sync_copy(z_v, acc.at[pl.ds(s * RPT + r * 16, 16)])
        return carry
    lax.fori_loop(0, RPT // 16, zc, 0)

    @pl.when(s == NS - 1)
    def _():
        pltpu.sync_copy(z_v, acc.at[pl.ds(NS * RPT, TAIL)])

    def ob(j, carry):
        for k in range(8):
            ones_v[j, pl.ds(k * 16, 16)] = jnp.ones((16,), jnp.float32)
        return carry
    lax.fori_loop(0, B, ob, 0)

    plsc.subcore_barrier()

    def step(i, carry):
        pltpu.sync_copy(dstr.at[c, s, i], dst_v)
        pltpu.sync_copy(ones_v, acc.at[dst_v.at[0]], add=True)
        return carry
    lax.fori_loop(0, IT, step, 0)

    plsc.subcore_barrier()

    def co(r, carry):
        pltpu.sync_copy(acc.at[pl.ds(s * RPT + r * 16, 16)], z_v)
        pltpu.sync_copy(z_v, out.at[c, pl.ds(s * RPT + r * 16, 16)])
        return carry
    lax.fori_loop(0, RPT // 16, co, 0)

    @pl.when(s == NS - 1)
    def _():
        pltpu.sync_copy(acc.at[pl.ds(NS * RPT, TAIL)], z_v)
        pltpu.sync_copy(z_v, out.at[c, pl.ds(NS * RPT, TAIL)])


_deg_pass = pl.kernel(
    _deg_body,
    out_type=jax.ShapeDtypeStruct((NC, N, F), jnp.float32),
    mesh=plsc.VectorSubcoreMesh(**_MESH),
    scratch_types=[
        pltpu.VMEM((1, B), jnp.int32),
        pltpu.VMEM((B, F), jnp.float32),
        pltpu.VMEM((16, F), jnp.float32),
        pltpu.VMEM_SHARED((N, F), jnp.float32),
    ],
)


R = 1000


def _dis(d0_ref, d1_ref):
    deg = d0_ref[...] + d1_ref[...] + 1.0
    return lax.rsqrt(deg)


def _elu(x):
    return jnp.where(x > 0, x, jnp.exp(jnp.minimum(x, 0.0)) - 1.0)


def _tc_first_body(x_ref, w_ref, d0_ref, d1_ref, o_ref):
    dis = _dis(d0_ref, d1_ref)
    o_ref[...] = dis * jnp.dot(x_ref[...], w_ref[...],
                               preferred_element_type=jnp.float32)


def _tc_mid_body(agg_ref, hs_ref, d0_ref, d1_ref, w_ref, b_ref, o_ref):
    dis = _dis(d0_ref, d1_ref)
    pre = (agg_ref[0] + agg_ref[1] + hs_ref[...]) * dis + b_ref[...]
    h = _elu(pre)
    o_ref[...] = dis * jnp.dot(h, w_ref[...],
                               preferred_element_type=jnp.float32)


def _tc_last_body(agg_ref, hs_ref, d0_ref, d1_ref, b_ref, wl_ref, bl_ref,
                  o_ref):
    dis = _dis(d0_ref, d1_ref)
    pre = (agg_ref[0] + agg_ref[1] + hs_ref[...]) * dis + b_ref[...]
    h = _elu(pre)
    o_ref[...] = jnp.dot(h, wl_ref[...],
                         preferred_element_type=jnp.float32) + bl_ref[...]


_row_spec = pl.BlockSpec((R, F), lambda i: (i, 0))
_agg_spec = pl.BlockSpec((NC, R, F), lambda i: (0, i, 0))
_deg_spec = pl.BlockSpec((R, 1), lambda i: (i, 0))
_w_spec = pl.BlockSpec((F, F), lambda i: (0, 0))
_b_spec = pl.BlockSpec((1, F), lambda i: (0, 0))
_wl_spec = pl.BlockSpec((F, 1), lambda i: (0, 0))
_bl_spec = pl.BlockSpec((1, 1), lambda i: (0, 0))
_o1_spec = pl.BlockSpec((R, 1), lambda i: (i, 0))

_tc_first = pl.pallas_call(
    _tc_first_body,
    grid=(N // R,),
    in_specs=[_row_spec, _w_spec, _deg_spec, _deg_spec],
    out_specs=_row_spec,
    out_shape=jax.ShapeDtypeStruct((N, F), jnp.float32),
)

_tc_mid = pl.pallas_call(
    _tc_mid_body,
    grid=(N // R,),
    in_specs=[_agg_spec, _row_spec, _deg_spec, _deg_spec, _w_spec, _b_spec],
    out_specs=_row_spec,
    out_shape=jax.ShapeDtypeStruct((N, F), jnp.float32),
)

_tc_last = pl.pallas_call(
    _tc_last_body,
    grid=(N // R,),
    in_specs=[_agg_spec, _row_spec, _deg_spec, _deg_spec, _b_spec, _wl_spec,
              _bl_spec],
    out_specs=_o1_spec,
    out_shape=jax.ShapeDtypeStruct((N, 1), jnp.float32),
)


@jax.jit
def kernel(x, edge_index, W0, b0, W1, b1, W2, b2, Wl, bl):
    src = edge_index[0].astype(jnp.int32).reshape(NC, NS, IT, 1, B)
    dst = edge_index[1].astype(jnp.int32).reshape(NC, NS, IT, 1, B)

    degp = _deg_pass(dst)
    d0 = degp[0, :, 0:1]
    d1 = degp[1, :, 0:1]

    hs = _tc_first(x, W0, d0, d1)
    agg = _edge_pass(hs, src, dst)
    hs = _tc_mid(agg, hs, d0, d1, W1, b0.reshape(1, F))
    agg = _edge_pass(hs, src, dst)
    hs = _tc_mid(agg, hs, d0, d1, W2, b1.reshape(1, F))
    agg = _edge_pass(hs, src, dst)
    out = _tc_last(agg, hs, d0, d1, b2.reshape(1, F), Wl,
                   bl.reshape(1, 1))
    return out


# speedup vs baseline: 20.2308x; 1.9904x over previous
import functools

import jax
import jax.numpy as jnp
from jax import lax
from jax.experimental import pallas as pl
from jax.experimental.pallas import tpu as pltpu
from jax.experimental.pallas import tpu_sc as plsc

N = 10000
F = 128
E = 320000
NC, NS = 2, 16
B = 125
EPT = E // (NC * NS)
IT = EPT // B
RPT = 624
TAIL = N - NS * RPT

_MESH = dict(core_axis_name="c", subcore_axis_name="s", num_cores=NC,
             num_subcores=NS)


def _edge_body(hs, srcr, dstr, out, si_a, si_b, di_v, rows_a, rows_b, z_v,
               acc, sg_a, sg_b, ss_a, ss_b):
    c = lax.axis_index("c")
    s = lax.axis_index("s")

    def zb(j, carry):
        for k in range(8):
            z_v[j, pl.ds(k * 16, 16)] = jnp.zeros((16,), jnp.float32)
        return carry
    lax.fori_loop(0, 16, zb, 0)

    def zc(r, carry):
        pltpu.sync_copy(z_v, acc.at[pl.ds(s * RPT + r * 16, 16)])
        return carry
    lax.fori_loop(0, RPT // 16, zc, 0)

    @pl.when(s == NS - 1)
    def _():
        pltpu.sync_copy(z_v, acc.at[pl.ds(NS * RPT, TAIL)])

    plsc.subcore_barrier()

    sidx = (si_a, si_b)
    rows = (rows_a, rows_b)
    gsem = (sg_a, sg_b)
    isem = (ss_a, ss_b)

    def wait_gather(p):
        pltpu.make_async_copy(hs.at[sidx[p].at[0]], rows[p], gsem[p]).wait()

    def wait_idx(q):
        pltpu.make_async_copy(srcr.at[0, 0, 0], sidx[q], isem[q]).wait()

    def scatter(i, p):
        pltpu.sync_copy(dstr.at[c, s, i], di_v)
        pltpu.sync_copy(rows[p], acc.at[di_v.at[0]], add=True)

    pltpu.sync_copy(srcr.at[c, s, 0], si_a)
    pltpu.async_copy(hs.at[si_a.at[0]], rows_a, sg_a)
    pltpu.async_copy(srcr.at[c, s, 1], si_b, ss_b)

    def step(i, carry):
        for p in (0, 1):
            ii = 2 * i + p
            q = 1 - p
            wait_gather(p)
            wait_idx(q)
            pltpu.async_copy(hs.at[sidx[q].at[0]], rows[q], gsem[q])
            pltpu.async_copy(srcr.at[c, s, ii + 2], sidx[p], isem[p])
            scatter(ii, p)
        return carry
    lax.fori_loop(0, (IT - 2) // 2, step, 0)

    wait_gather(0)
    wait_idx(1)
    pltpu.async_copy(hs.at[si_b.at[0]], rows_b, sg_b)
    scatter(IT - 2, 0)
    wait_gather(1)
    scatter(IT - 1, 1)

    plsc.subcore_barrier()

    def co(r, carry):
        pltpu.sync_copy(acc.at[pl.ds(s * RPT + r * 16, 16)], z_v)
        pltpu.sync_copy(z_v, out.at[c, pl.ds(s * RPT + r * 16, 16)])
        return carry
    lax.fori_loop(0, RPT // 16, co, 0)

    @pl.when(s == NS - 1)
    def _():
        pltpu.sync_copy(acc.at[pl.ds(NS * RPT, TAIL)], z_v)
        pltpu.sync_copy(z_v, out.at[c, pl.ds(NS * RPT, TAIL)])


_edge_pass = pl.kernel(
    _edge_body,
    out_type=jax.ShapeDtypeStruct((NC, N, F), jnp.float32),
    mesh=plsc.VectorSubcoreMesh(**_MESH),
    scratch_types=[
        pltpu.VMEM((1, B), jnp.int32),
        pltpu.VMEM((1, B), jnp.int32),
        pltpu.VMEM((1, B), jnp.int32),
        pltpu.VMEM((B, F), jnp.float32),
        pltpu.VMEM((B, F), jnp.float32),
        pltpu.VMEM((16, F), jnp.float32),
        pltpu.VMEM_SHARED((N, F), jnp.float32),
        pltpu.SemaphoreType.DMA,
        pltpu.SemaphoreType.DMA,
        pltpu.SemaphoreType.DMA,
        pltpu.SemaphoreType.DMA,
    ],
)


def _deg_body(dstr, out, dst_v, ones_v, z_v, acc):
    c = lax.axis_index("c")
    s = lax.axis_index("s")

    def zb(j, carry):
        for k in range(8):
            z_v[j, pl.ds(k * 16, 16)] = jnp.zeros((16,), jnp.float32)
        return carry
    lax.fori_loop(0, 16, zb, 0)

    def zc(r, carry):
        pltpu.sync_copy(z_v, acc.at[pl.ds(s * RPT + r * 16, 16)])
        return carry
    lax.fori_loop(0, RPT // 16, zc, 0)

    @pl.when(s == NS - 1)
    def _():
        pltpu.sync_copy(z_v, acc.at[pl.ds(NS * RPT, TAIL)])

    def ob(j, carry):
        for k in range(8):
            ones_v[j, pl.ds(k * 16, 16)] = jnp.ones((16,), jnp.float32)
        return carry
    lax.fori_loop(0, B, ob, 0)

    plsc.subcore_barrier()

    def step(i, carry):
        pltpu.sync_copy(dstr.at[c, s, i], dst_v)
        pltpu.sync_copy(ones_v, acc.at[dst_v.at[0]], add=True)
        return carry
    lax.fori_loop(0, IT, step, 0)

    plsc.subcore_barrier()

    def co(r, carry):
        pltpu.sync_copy(acc.at[pl.ds(s * RPT + r * 16, 16)], z_v)
        pltpu.sync_copy(z_v, out.at[c, pl.ds(s * RPT + r * 16, 16)])
        return carry
    lax.fori_loop(0, RPT // 16, co, 0)

    @pl.when(s == NS - 1)
    def _():
        pltpu.sync_copy(acc.at[pl.ds(NS * RPT, TAIL)], z_v)
        pltpu.sync_copy(z_v, out.at[c, pl.ds(NS * RPT, TAIL)])


_deg_pass = pl.kernel(
    _deg_body,
    out_type=jax.ShapeDtypeStruct((NC, N, F), jnp.float32),
    mesh=plsc.VectorSubcoreMesh(**_MESH),
    scratch_types=[
        pltpu.VMEM((1, B), jnp.int32),
        pltpu.VMEM((B, F), jnp.float32),
        pltpu.VMEM((16, F), jnp.float32),
        pltpu.VMEM_SHARED((N, F), jnp.float32),
    ],
)


R = 1000


def _dis(d0_ref, d1_ref):
    deg = d0_ref[...] + d1_ref[...] + 1.0
    return lax.rsqrt(deg)


def _elu(x):
    return jnp.where(x > 0, x, jnp.exp(jnp.minimum(x, 0.0)) - 1.0)


def _tc_first_body(x_ref, w_ref, d0_ref, d1_ref, o_ref):
    dis = _dis(d0_ref, d1_ref)
    o_ref[...] = dis * jnp.dot(x_ref[...], w_ref[...],
                               preferred_element_type=jnp.float32)


def _tc_mid_body(agg_ref, hs_ref, d0_ref, d1_ref, w_ref, b_ref, o_ref):
    dis = _dis(d0_ref, d1_ref)
    pre = (agg_ref[0] + agg_ref[1] + hs_ref[...]) * dis + b_ref[...]
    h = _elu(pre)
    o_ref[...] = dis * jnp.dot(h, w_ref[...],
                               preferred_element_type=jnp.float32)


def _tc_last_body(agg_ref, hs_ref, d0_ref, d1_ref, b_ref, wl_ref, bl_ref,
                  o_ref):
    dis = _dis(d0_ref, d1_ref)
    pre = (agg_ref[0] + agg_ref[1] + hs_ref[...]) * dis + b_ref[...]
    h = _elu(pre)
    o_ref[...] = jnp.dot(h, wl_ref[...],
                         preferred_element_type=jnp.float32) + bl_ref[...]


_row_spec = pl.BlockSpec((R, F), lambda i: (i, 0))
_agg_spec = pl.BlockSpec((NC, R, F), lambda i: (0, i, 0))
_deg_spec = pl.BlockSpec((R, 1), lambda i: (i, 0))
_w_spec = pl.BlockSpec((F, F), lambda i: (0, 0))
_b_spec = pl.BlockSpec((1, F), lambda i: (0, 0))
_wl_spec = pl.BlockSpec((F, 1), lambda i: (0, 0))
_bl_spec = pl.BlockSpec((1, 1), lambda i: (0, 0))
_o1_spec = pl.BlockSpec((R, 1), lambda i: (i, 0))

_tc_first = pl.pallas_call(
    _tc_first_body,
    grid=(N // R,),
    in_specs=[_row_spec, _w_spec, _deg_spec, _deg_spec],
    out_specs=_row_spec,
    out_shape=jax.ShapeDtypeStruct((N, F), jnp.float32),
)

_tc_mid = pl.pallas_call(
    _tc_mid_body,
    grid=(N // R,),
    in_specs=[_agg_spec, _row_spec, _deg_spec, _deg_spec, _w_spec, _b_spec],
    out_specs=_row_spec,
    out_shape=jax.ShapeDtypeStruct((N, F), jnp.float32),
)

_tc_last = pl.pallas_call(
    _tc_last_body,
    grid=(N // R,),
    in_specs=[_agg_spec, _row_spec, _deg_spec, _deg_spec, _b_spec, _wl_spec,
              _bl_spec],
    out_specs=_o1_spec,
    out_shape=jax.ShapeDtypeStruct((N, 1), jnp.float32),
)


@jax.jit
def kernel(x, edge_index, W0, b0, W1, b1, W2, b2, Wl, bl):
    src = edge_index[0].astype(jnp.int32).reshape(NC, NS, IT, 1, B)
    dst = edge_index[1].astype(jnp.int32).reshape(NC, NS, IT, 1, B)

    degp = _deg_pass(dst)
    d0 = degp[0, :, 0:1]
    d1 = degp[1, :, 0:1]

    hs = _tc_first(x, W0, d0, d1)
    agg = _edge_pass(hs, src, dst)
    hs = _tc_mid(agg, hs, d0, d1, W1, b0.reshape(1, F))
    agg = _edge_pass(hs, src, dst)
    hs = _tc_mid(agg, hs, d0, d1, W2, b1.reshape(1, F))
    agg = _edge_pass(hs, src, dst)
    out = _tc_last(agg, hs, d0, d1, b2.reshape(1, F), Wl,
                   bl.reshape(1, 1))
    return out


# speedup vs baseline: 20.2727x; 1.0021x over previous
import functools

import jax
import jax.numpy as jnp
from jax import lax
from jax.experimental import pallas as pl
from jax.experimental.pallas import tpu as pltpu
from jax.experimental.pallas import tpu_sc as plsc

N = 10000
F = 128
E = 320000
NC, NS = 2, 16
B = 125
EPT = E // (NC * NS)
IT = EPT // B
RPT = 624
TAIL = N - NS * RPT

_MESH = dict(core_axis_name="c", subcore_axis_name="s", num_cores=NC,
             num_subcores=NS)


def _edge_body(hs, srcr, dstr, out, si_a, si_b, di_a, di_b, rows_a, rows_b,
               z_a, z_b, acc, sg_a, sg_b, ss_a, ss_b, sd_a, sd_b, zs_a, zs_b):
    c = lax.axis_index("c")
    s = lax.axis_index("s")

    def zb(j, carry):
        for k in range(8):
            z_a[j, pl.ds(k * 16, 16)] = jnp.zeros((16,), jnp.float32)
        return carry
    lax.fori_loop(0, 16, zb, 0)

    def zc(r, carry):
        pltpu.sync_copy(z_a, acc.at[pl.ds(s * RPT + r * 16, 16)])
        return carry
    lax.fori_loop(0, RPT // 16, zc, 0)

    @pl.when(s == NS - 1)
    def _():
        pltpu.sync_copy(z_a, acc.at[pl.ds(NS * RPT, TAIL)])

    plsc.subcore_barrier()

    sidx = (si_a, si_b)
    didx = (di_a, di_b)
    rows = (rows_a, rows_b)
    gsem = (sg_a, sg_b)
    isem = (ss_a, ss_b)
    dsem = (sd_a, sd_b)

    def wait_gather(p):
        pltpu.make_async_copy(hs.at[sidx[p].at[0]], rows[p], gsem[p]).wait()

    def wait_idx(q):
        pltpu.make_async_copy(srcr.at[0, 0, 0], sidx[q], isem[q]).wait()

    def wait_didx(p):
        pltpu.make_async_copy(dstr.at[0, 0, 0], didx[p], dsem[p]).wait()

    pltpu.sync_copy(srcr.at[c, s, 0], si_a)
    pltpu.async_copy(dstr.at[c, s, 0], di_a, sd_a)
    pltpu.async_copy(hs.at[si_a.at[0]], rows_a, sg_a)
    pltpu.async_copy(srcr.at[c, s, 1], si_b, ss_b)
    pltpu.async_copy(dstr.at[c, s, 1], di_b, sd_b)

    def step(i, carry):
        for p in (0, 1):
            ii = 2 * i + p
            q = 1 - p
            wait_gather(p)
            wait_idx(q)
            pltpu.async_copy(hs.at[sidx[q].at[0]], rows[q], gsem[q])
            pltpu.async_copy(srcr.at[c, s, ii + 2], sidx[p], isem[p])
            wait_didx(p)
            pltpu.sync_copy(rows[p], acc.at[didx[p].at[0]], add=True)
            pltpu.async_copy(dstr.at[c, s, ii + 2], didx[p], dsem[p])
        return carry
    lax.fori_loop(0, (IT - 2) // 2, step, 0)

    wait_gather(0)
    wait_idx(1)
    pltpu.async_copy(hs.at[si_b.at[0]], rows_b, sg_b)
    wait_didx(0)
    pltpu.sync_copy(rows_a, acc.at[di_a.at[0]], add=True)
    wait_gather(1)
    wait_didx(1)
    pltpu.sync_copy(rows_b, acc.at[di_b.at[0]], add=True)

    plsc.subcore_barrier()

    def co(r, carry):
        pltpu.sync_copy(acc.at[pl.ds(s * RPT + r * 16, 16)], z_a)
        pltpu.sync_copy(z_a, out.at[c, pl.ds(s * RPT + r * 16, 16)])
        return carry
    lax.fori_loop(0, RPT // 16, co, 0)

    @pl.when(s == NS - 1)
    def _():
        pltpu.sync_copy(acc.at[pl.ds(NS * RPT, TAIL)], z_a)
        pltpu.sync_copy(z_a, out.at[c, pl.ds(NS * RPT, TAIL)])


_edge_pass = pl.kernel(
    _edge_body,
    out_type=jax.ShapeDtypeStruct((NC, N, F), jnp.float32),
    mesh=plsc.VectorSubcoreMesh(**_MESH),
    scratch_types=[
        pltpu.VMEM((1, B), jnp.int32),
        pltpu.VMEM((1, B), jnp.int32),
        pltpu.VMEM((1, B), jnp.int32),
        pltpu.VMEM((1, B), jnp.int32),
        pltpu.VMEM((B, F), jnp.float32),
        pltpu.VMEM((B, F), jnp.float32),
        pltpu.VMEM((16, F), jnp.float32),
        pltpu.VMEM((16, F), jnp.float32),
        pltpu.VMEM_SHARED((N, F), jnp.float32),
        pltpu.SemaphoreType.DMA,
        pltpu.SemaphoreType.DMA,
        pltpu.SemaphoreType.DMA,
        pltpu.SemaphoreType.DMA,
        pltpu.SemaphoreType.DMA,
        pltpu.SemaphoreType.DMA,
        pltpu.SemaphoreType.DMA,
        pltpu.SemaphoreType.DMA,
    ],
)


def _deg_body(dstr, out, dst_v, ones_v, z_v, acc):
    c = lax.axis_index("c")
    s = lax.axis_index("s")

    def zb(j, carry):
        for k in range(8):
            z_v[j, pl.ds(k * 16, 16)] = jnp.zeros((16,), jnp.float32)
        return carry
    lax.fori_loop(0, 16, zb, 0)

    def zc(r, carry):
        pltpu.sync_copy(z_v, acc.at[pl.ds(s * RPT + r * 16, 16)])
        return carry
    lax.fori_loop(0, RPT // 16, zc, 0)

    @pl.when(s == NS - 1)
    def _():
        pltpu.sync_copy(z_v, acc.at[pl.ds(NS * RPT, TAIL)])

    def ob(j, carry):
        for k in range(8):
            ones_v[j, pl.ds(k * 16, 16)] = jnp.ones((16,), jnp.float32)
        return carry
    lax.fori_loop(0, B, ob, 0)

    plsc.subcore_barrier()

    def step(i, carry):
        pltpu.sync_copy(dstr.at[c, s, i], dst_v)
        pltpu.sync_copy(ones_v, acc.at[dst_v.at[0]], add=True)
        return carry
    lax.fori_loop(0, IT, step, 0)

    plsc.subcore_barrier()

    def co(r, carry):
        pltpu.sync_copy(acc.at[pl.ds(s * RPT + r * 16, 16)], z_v)
        pltpu.sync_copy(z_v, out.at[c, pl.ds(s * RPT + r * 16, 16)])
        return carry
    lax.fori_loop(0, RPT // 16, co, 0)

    @pl.when(s == NS - 1)
    def _():
        pltpu.sync_copy(acc.at[pl.ds(NS * RPT, TAIL)], z_v)
        pltpu.sync_copy(z_v, out.at[c, pl.ds(NS * RPT, TAIL)])


_deg_pass = pl.kernel(
    _deg_body,
    out_type=jax.ShapeDtypeStruct((NC, N, F), jnp.float32),
    mesh=plsc.VectorSubcoreMesh(**_MESH),
    scratch_types=[
        pltpu.VMEM((1, B), jnp.int32),
        pltpu.VMEM((B, F), jnp.float32),
        pltpu.VMEM((16, F), jnp.float32),
        pltpu.VMEM_SHARED((N, F), jnp.float32),
    ],
)


R = 1000


def _dis(d0_ref, d1_ref):
    deg = d0_ref[...] + d1_ref[...] + 1.0
    return lax.rsqrt(deg)


def _elu(x):
    return jnp.where(x > 0, x, jnp.exp(jnp.minimum(x, 0.0)) - 1.0)


def _tc_first_body(x_ref, w_ref, d0_ref, d1_ref, o_ref):
    dis = _dis(d0_ref, d1_ref)
    o_ref[...] = dis * jnp.dot(x_ref[...], w_ref[...],
                               preferred_element_type=jnp.float32)


def _tc_mid_body(agg_ref, hs_ref, d0_ref, d1_ref, w_ref, b_ref, o_ref):
    dis = _dis(d0_ref, d1_ref)
    pre = (agg_ref[0] + agg_ref[1] + hs_ref[...]) * dis + b_ref[...]
    h = _elu(pre)
    o_ref[...] = dis * jnp.dot(h, w_ref[...],
                               preferred_element_type=jnp.float32)


def _tc_last_body(agg_ref, hs_ref, d0_ref, d1_ref, b_ref, wl_ref, bl_ref,
                  o_ref):
    dis = _dis(d0_ref, d1_ref)
    pre = (agg_ref[0] + agg_ref[1] + hs_ref[...]) * dis + b_ref[...]
    h = _elu(pre)
    o_ref[...] = jnp.dot(h, wl_ref[...],
                         preferred_element_type=jnp.float32) + bl_ref[...]


_row_spec = pl.BlockSpec((R, F), lambda i: (i, 0))
_agg_spec = pl.BlockSpec((NC, R, F), lambda i: (0, i, 0))
_deg_spec = pl.BlockSpec((R, 1), lambda i: (i, 0))
_w_spec = pl.BlockSpec((F, F), lambda i: (0, 0))
_b_spec = pl.BlockSpec((1, F), lambda i: (0, 0))
_wl_spec = pl.BlockSpec((F, 1), lambda i: (0, 0))
_bl_spec = pl.BlockSpec((1, 1), lambda i: (0, 0))
_o1_spec = pl.BlockSpec((R, 1), lambda i: (i, 0))

_tc_first = pl.pallas_call(
    _tc_first_body,
    grid=(N // R,),
    in_specs=[_row_spec, _w_spec, _deg_spec, _deg_spec],
    out_specs=_row_spec,
    out_shape=jax.ShapeDtypeStruct((N, F), jnp.float32),
)

_tc_mid = pl.pallas_call(
    _tc_mid_body,
    grid=(N // R,),
    in_specs=[_agg_spec, _row_spec, _deg_spec, _deg_spec, _w_spec, _b_spec],
    out_specs=_row_spec,
    out_shape=jax.ShapeDtypeStruct((N, F), jnp.float32),
)

_tc_last = pl.pallas_call(
    _tc_last_body,
    grid=(N // R,),
    in_specs=[_agg_spec, _row_spec, _deg_spec, _deg_spec, _b_spec, _wl_spec,
              _bl_spec],
    out_specs=_o1_spec,
    out_shape=jax.ShapeDtypeStruct((N, 1), jnp.float32),
)


@jax.jit
def kernel(x, edge_index, W0, b0, W1, b1, W2, b2, Wl, bl):
    src = edge_index[0].astype(jnp.int32).reshape(NC, NS, IT, 1, B)
    dst = edge_index[1].astype(jnp.int32).reshape(NC, NS, IT, 1, B)

    degp = _deg_pass(dst)
    d0 = degp[0, :, 0:1]
    d1 = degp[1, :, 0:1]

    hs = _tc_first(x, W0, d0, d1)
    agg = _edge_pass(hs, src, dst)
    hs = _tc_mid(agg, hs, d0, d1, W1, b0.reshape(1, F))
    agg = _edge_pass(hs, src, dst)
    hs = _tc_mid(agg, hs, d0, d1, W2, b1.reshape(1, F))
    agg = _edge_pass(hs, src, dst)
    out = _tc_last(agg, hs, d0, d1, b2.reshape(1, F), Wl,
                   bl.reshape(1, 1))
    return out
